# baseline (speedup 1.0000x reference)
import functools

import jax
import jax.numpy as jnp
from jax import lax
from jax.experimental import pallas as pl
from jax.experimental.pallas import tpu as pltpu
from jax.experimental.pallas import tpu_sc as plsc

EMB = 300
HID = 150
N_NODES = 10000
N_EDGES = 320000
DPAD = 152
NWORKERS = 32
EPW = N_EDGES // NWORKERS
KCH = 80
NCH = EPW // KCH

TPAD = 304
N_LOOKUP = 9600 + 20000
GCH = 12
N_LOOKUP_PAD = NWORKERS * GCH * KCH
LPW = GCH * KCH

_SC_MESH = plsc.VectorSubcoreMesh(core_axis_name="c", subcore_axis_name="s")


@functools.partial(
    pl.kernel,
    out_type=jax.ShapeDtypeStruct((2, N_NODES), jnp.float32),
    mesh=_SC_MESH,
    compiler_params=pltpu.CompilerParams(use_tc_tiling_on_sc=False),
    scratch_types=[
        pltpu.VMEM((NCH, KCH), jnp.int32),
        pltpu.VMEM((KCH,), jnp.float32),
        pltpu.VMEM_SHARED((N_NODES,), jnp.float32),
    ],
)
def _sc_degree(dst_hbm, zero_hbm, out_hbm, dst_v, ones_v, acc):
    c = lax.axis_index("c")
    s = lax.axis_index("s")
    wid = c * 16 + s

    @pl.when(s == 0)
    def _():
        pltpu.sync_copy(zero_hbm, acc)

    pltpu.sync_copy(dst_hbm.at[wid], dst_v)

    def fill(i, carry):
        ones_v[pl.ds(i * 16, 16)] = jnp.ones((16,), jnp.float32)
        return carry

    lax.fori_loop(0, KCH // 16, fill, 0)
    plsc.subcore_barrier()

    def body(j, carry):
        pltpu.sync_copy(ones_v, acc.at[dst_v.at[j]], add=True)
        return carry

    lax.fori_loop(0, NCH, body, 0)
    plsc.subcore_barrier()

    @pl.when(s == 0)
    def _():
        pltpu.sync_copy(acc, out_hbm.at[c])


@functools.partial(
    pl.kernel,
    out_type=jax.ShapeDtypeStruct((2, N_NODES, DPAD), jnp.float32),
    mesh=_SC_MESH,
    compiler_params=pltpu.CompilerParams(use_tc_tiling_on_sc=False),
    scratch_types=[
        pltpu.VMEM((NCH, KCH), jnp.int32),
        pltpu.VMEM((NCH, KCH), jnp.int32),
        pltpu.VMEM((KCH, DPAD), jnp.float32),
        pltpu.VMEM_SHARED((N_NODES, DPAD), jnp.float32),
    ],
)
def _sc_edge_prop(src_hbm, dst_hbm, y_hbm, zero_hbm, out_hbm,
                  src_v, dst_v, rows_v, acc):
    c = lax.axis_index("c")
    s = lax.axis_index("s")
    wid = c * 16 + s

    @pl.when(s == 0)
    def _():
        pltpu.sync_copy(zero_hbm, acc)

    pltpu.sync_copy(src_hbm.at[wid], src_v)
    pltpu.sync_copy(dst_hbm.at[wid], dst_v)
    plsc.subcore_barrier()

    def body(j, carry):
        pltpu.sync_copy(y_hbm.at[src_v.at[j]], rows_v)
        pltpu.sync_copy(rows_v, acc.at[dst_v.at[j]], add=True)
        return carry

    lax.fori_loop(0, NCH, body, 0)
    plsc.subcore_barrier()
    pltpu.sync_copy(acc.at[pl.ds(s * 624, 624)],
                    out_hbm.at[c, pl.ds(s * 624, 624)])

    @pl.when(s == 15)
    def _():
        pltpu.sync_copy(acc.at[pl.ds(9984, N_NODES - 9984)],
                        out_hbm.at[c, pl.ds(9984, N_NODES - 9984)])


@functools.partial(
    pl.kernel,
    out_type=jax.ShapeDtypeStruct((N_LOOKUP_PAD, TPAD), jnp.float32),
    mesh=_SC_MESH,
    compiler_params=pltpu.CompilerParams(use_tc_tiling_on_sc=False),
    scratch_types=[
        pltpu.VMEM((GCH, KCH), jnp.int32),
        pltpu.VMEM((KCH, TPAD), jnp.float32),
    ],
)
def _sc_embed_gather(table_hbm, idx_hbm, out_hbm, idx_v, rows_v):
    c = lax.axis_index("c")
    s = lax.axis_index("s")
    wid = c * 16 + s
    pltpu.sync_copy(idx_hbm.at[wid], idx_v)

    def body(j, carry):
        pltpu.sync_copy(table_hbm.at[idx_v.at[j]], rows_v)
        pltpu.sync_copy(rows_v, out_hbm.at[pl.ds(wid * LPW + j * KCH, KCH)])
        return carry

    lax.fori_loop(0, GCH, body, 0)


def _pad_table(tt):
    def kern(x_ref, o_ref):
        xt = jnp.transpose(x_ref[...], (1, 0))
        o_ref[...] = jnp.concatenate(
            [xt, jnp.zeros((xt.shape[0], TPAD - EMB), jnp.float32)], axis=1)

    return pl.pallas_call(
        kern,
        grid=(98,),
        in_specs=[pl.BlockSpec((EMB, 512), lambda i: (0, i))],
        out_specs=pl.BlockSpec((512, TPAD), lambda i: (i, 0)),
        out_shape=jax.ShapeDtypeStruct((tt.shape[1], TPAD), jnp.float32),
    )(tt)

def _mm_bias(x, wt, b):
    def kern(x_ref, wt_ref, b_ref, o_ref):
        o_ref[...] = jnp.dot(x_ref[...], wt_ref[...],
                             preferred_element_type=jnp.float32) + b_ref[...]

    return pl.pallas_call(
        kern,
        out_shape=jax.ShapeDtypeStruct((x.shape[0], wt.shape[1]), jnp.float32),
    )(x, wt, b)


def _mm_scale(x, wt, dinv):
    def kern(x_ref, wt_ref, d_ref, o_ref):
        o_ref[...] = jnp.dot(x_ref[...], wt_ref[...],
                             preferred_element_type=jnp.float32) * d_ref[...]

    return pl.pallas_call(
        kern,
        out_shape=jax.ShapeDtypeStruct((x.shape[0], wt.shape[1]), jnp.float32),
    )(x, wt, dinv)


def _dinv_col(deg_t):
    def kern(d_ref, o_ref):
        o_ref[...] = lax.rsqrt(1.0 + jnp.sum(d_ref[...], axis=1, keepdims=True))

    return pl.pallas_call(
        kern,
        out_shape=jax.ShapeDtypeStruct((deg_t.shape[0], 1), jnp.float32),
    )(deg_t)


def _epilogue(p, y, dinv, b, elu):
    def kern(p_ref, y_ref, d_ref, b_ref, o_ref):
        t = (p_ref[0] + p_ref[1] + y_ref[...]) * d_ref[...] + b_ref[...]
        if elu:
            t = jnp.where(t > 0, t, jnp.exp(jnp.minimum(t, 0.0)) - 1.0)
        o_ref[...] = t

    return pl.pallas_call(
        kern,
        out_shape=jax.ShapeDtypeStruct(y.shape, jnp.float32),
    )(p, y, dinv, b)


def _lstm_scan_kernel(xp, w, Bn, T):
    def kern(xp_ref, w_ref, hs_ref):
        w_v = w_ref[...]

        def step(t, carry):
            h, c = carry
            g = xp_ref[t] + jnp.dot(h, w_v, preferred_element_type=jnp.float32)
            i = jax.nn.sigmoid(g[:, 0:300])
            f = jax.nn.sigmoid(g[:, 300:600])
            gg = jnp.tanh(g[:, 600:900])
            o = jax.nn.sigmoid(g[:, 900:1200])
            c = f * c + i * gg
            h = o * jnp.tanh(c)
            hs_ref[t] = h
            return (h, c)

        z = jnp.zeros((Bn, 2 * HID), jnp.float32)
        lax.fori_loop(0, T, step, (z, z))

    return pl.pallas_call(
        kern,
        out_shape=jax.ShapeDtypeStruct((T, Bn, 2 * HID), jnp.float32),
    )(xp, w)


def _lstm_layer(p, layer, x, lens):
    Bn, T, _ = x.shape
    H = HID
    t = jnp.arange(T)
    idx = lens[:, None] - 1 - t[None, :]
    valid = (idx >= 0).astype(x.dtype)
    idxc = jnp.clip(idx, 0, T - 1)
    prev = (idxc[:, :, None] == t[None, None, :]).astype(jnp.float32)
    prev = prev * valid[:, :, None]
    xb = jnp.einsum('btl,ble->bte', prev, x)

    wih_f = p['lstm_wih_%df' % layer].T
    wih_b = p['lstm_wih_%db' % layer].T
    whh_f = p['lstm_whh_%df' % layer].T
    whh_b = p['lstm_whh_%db' % layer].T
    bias_f = p['lstm_bih_%df' % layer] + p['lstm_bhh_%df' % layer]
    bias_b = p['lstm_bih_%db' % layer] + p['lstm_bhh_%db' % layer]

    xpf = (x @ wih_f + bias_f).reshape(Bn, T, 4, H)
    xpb = (xb @ wih_b + bias_b).reshape(Bn, T, 4, H)
    xp = jnp.concatenate([xpf, xpb], axis=-1).reshape(Bn, T, 8 * H)
    xp = jnp.swapaxes(xp, 0, 1)

    w = jnp.zeros((2 * H, 8 * H), jnp.float32)
    for k in range(4):
        w = w.at[:H, k * 2 * H:k * 2 * H + H].set(whh_f[:, k * H:(k + 1) * H])
        w = w.at[H:, k * 2 * H + H:(k + 1) * 2 * H].set(whh_b[:, k * H:(k + 1) * H])

    hs = _lstm_scan_kernel(xp, w, Bn, T)
    fwd = jnp.swapaxes(hs[:, :, :H], 0, 1)
    bwd = jnp.swapaxes(hs[:, :, H:], 0, 1)
    bwd = jnp.einsum('btl,ble->bte', prev, bwd)
    mask = (t[None, :] < lens[:, None]).astype(hs.dtype)[:, :, None]
    return jnp.concatenate([fwd * mask, bwd * mask], axis=-1)


def _bilstm(p, x, lens):
    h = x
    for layer in (0, 1):
        h = _lstm_layer(p, layer, h, lens)
    return h


def _encoder(p, pos_table, sent_pos_table, x, sent_x, embed, emb_flat):
    Bn, Sn, Ln = sent_x.shape
    BS = Bn * Sn
    tokens = sent_x.reshape(BS, Ln)
    sentlen = jnp.sum(tokens != 0, axis=-1)
    pos_idx = jnp.arange(1, Ln + 1)[None, :]
    posmask = (pos_idx <= sentlen[:, None]).astype(jnp.float32)
    pos_emb = posmask[:, :, None] * pos_table[1:Ln + 1][None, :, :]
    conv_in = emb_flat + pos_emb
    feats = []
    for h in range(2, 8):
        w = p['conv_w_%d' % h]
        bb = p['conv_b_%d' % h]
        P = Ln - h + 1
        acc = None
        for dh in range(h):
            term = jnp.einsum('bpe,oe->bpo', conv_in[:, dh:dh + P, :],
                              w[:, 0, dh, :])
            acc = term if acc is None else acc + term
        out = jax.nn.relu(acc + bb[None, None, :])
        feats.append(jnp.max(out, axis=1))
    ngram = jnp.concatenate(feats, axis=1)
    spos = jnp.tile(sent_pos_table[:Sn], (Bn, 1))
    cnn_feature = (ngram + spos) @ p['cnn_proj_w'].T + p['cnn_proj_b']
    cnn_feature = cnn_feature.reshape(Bn, Sn, HID)
    glen = jnp.sum(sent_x[:, :, 0] != 0, axis=-1)
    lstm_out = _bilstm(p, cnn_feature, glen)
    lstm_feature = lstm_out @ p['lstm_proj_w'].T + p['lstm_proj_b']
    sent_feat = jnp.concatenate([cnn_feature, lstm_feature], axis=-1)
    xcat = jnp.concatenate([embed, sent_feat], axis=1)
    return xcat.reshape(-1, EMB), embed


def kernel(x, sent_x, edge_index, params, pos_table, sent_pos_table):
    p = params

    src_r = edge_index[0].reshape(NWORKERS, NCH, KCH)
    dst_r = edge_index[1].reshape(NWORKERS, NCH, KCH)
    zero_deg = jnp.zeros((N_NODES,), jnp.float32)
    zero_rows = jnp.zeros((N_NODES, DPAD), jnp.float32)

    deg_part = _sc_degree(dst_r, zero_deg)
    dinv = _dinv_col(deg_part.T)

    tpad = _pad_table(p['embed'].T)
    all_idx = jnp.concatenate([
        x.reshape(-1), sent_x.reshape(-1),
        jnp.arange(N_LOOKUP_PAD - N_LOOKUP, dtype=jnp.int32)])
    g = _sc_embed_gather(tpad, all_idx.reshape(NWORKERS, GCH, KCH))
    embed = g[:9600, :EMB].reshape(x.shape[0], x.shape[1], EMB)
    emb_flat = g[9600:N_LOOKUP, :EMB].reshape(-1, sent_x.shape[2], EMB)

    xcat, embed = _encoder(p, pos_table, sent_pos_table, x, sent_x,
                           embed, emb_flat)

    h0 = _mm_bias(xcat, p['gc0_w'].T, p['gc0_b'][None, :])

    w1t = jnp.zeros((HID, DPAD), jnp.float32).at[:, :HID].set(p['gc1_w'].T)
    w2t = jnp.zeros((DPAD, DPAD), jnp.float32).at[:HID, :HID].set(p['gc2_w'].T)
    b1 = jnp.zeros((1, DPAD), jnp.float32).at[0, :HID].set(p['gc1_b'])
    b2 = jnp.zeros((1, DPAD), jnp.float32).at[0, :HID].set(p['gc2_b'])

    y1 = _mm_scale(h0, w1t, dinv)
    p1 = _sc_edge_prop(src_r, dst_r, y1, zero_rows)
    h1 = _epilogue(p1, y1, dinv, b1, elu=True)

    y2 = _mm_scale(h1, w2t, dinv)
    p2 = _sc_edge_prop(src_r, dst_r, y2, zero_rows)
    out = _epilogue(p2, y2, dinv, b2, elu=False)

    return out[:, :HID], embed


# speedup vs baseline: 1.0321x; 1.0321x over previous
import functools

import jax
import jax.numpy as jnp
from jax import lax
from jax.experimental import pallas as pl
from jax.experimental.pallas import tpu as pltpu
from jax.experimental.pallas import tpu_sc as plsc

EMB = 300
HID = 150
N_NODES = 10000
N_EDGES = 320000
DPAD = 152
NWORKERS = 32
EPW = N_EDGES // NWORKERS
KCH = 80
NCH = EPW // KCH

TPAD = 304
N_LOOKUP = 9600 + 20000
GCH = 12
N_LOOKUP_PAD = NWORKERS * GCH * KCH
LPW = GCH * KCH

_SC_MESH = plsc.VectorSubcoreMesh(core_axis_name="c", subcore_axis_name="s")


@functools.partial(
    pl.kernel,
    out_type=jax.ShapeDtypeStruct((2, N_NODES), jnp.float32),
    mesh=_SC_MESH,
    compiler_params=pltpu.CompilerParams(use_tc_tiling_on_sc=False),
    scratch_types=[
        pltpu.VMEM((NCH, KCH), jnp.int32),
        pltpu.VMEM((KCH,), jnp.float32),
        pltpu.VMEM_SHARED((N_NODES,), jnp.float32),
    ],
)
def _sc_degree(dst_hbm, zero_hbm, out_hbm, dst_v, ones_v, acc):
    c = lax.axis_index("c")
    s = lax.axis_index("s")
    wid = c * 16 + s

    @pl.when(s == 0)
    def _():
        pltpu.sync_copy(zero_hbm, acc)

    pltpu.sync_copy(dst_hbm.at[wid], dst_v)

    def fill(i, carry):
        ones_v[pl.ds(i * 16, 16)] = jnp.ones((16,), jnp.float32)
        return carry

    lax.fori_loop(0, KCH // 16, fill, 0)
    plsc.subcore_barrier()

    def body(j, carry):
        pltpu.sync_copy(ones_v, acc.at[dst_v.at[j]], add=True)
        return carry

    lax.fori_loop(0, NCH, body, 0)
    plsc.subcore_barrier()

    @pl.when(s == 0)
    def _():
        pltpu.sync_copy(acc, out_hbm.at[c])


@functools.partial(
    pl.kernel,
    out_type=jax.ShapeDtypeStruct((2, N_NODES, DPAD), jnp.float32),
    mesh=_SC_MESH,
    compiler_params=pltpu.CompilerParams(use_tc_tiling_on_sc=False),
    scratch_types=[
        pltpu.VMEM((NCH, KCH), jnp.int32),
        pltpu.VMEM((NCH, KCH), jnp.int32),
        pltpu.VMEM((KCH, DPAD), jnp.float32),
        pltpu.VMEM_SHARED((N_NODES, DPAD), jnp.float32),
    ],
)
def _sc_edge_prop(src_hbm, dst_hbm, y_hbm, zero_hbm, out_hbm,
                  src_v, dst_v, rows_v, acc):
    c = lax.axis_index("c")
    s = lax.axis_index("s")
    wid = c * 16 + s

    @pl.when(s == 0)
    def _():
        pltpu.sync_copy(zero_hbm, acc)

    pltpu.sync_copy(src_hbm.at[wid], src_v)
    pltpu.sync_copy(dst_hbm.at[wid], dst_v)
    plsc.subcore_barrier()

    def body(j, carry):
        pltpu.sync_copy(y_hbm.at[src_v.at[j]], rows_v)
        pltpu.sync_copy(rows_v, acc.at[dst_v.at[j]], add=True)
        return carry

    lax.fori_loop(0, NCH, body, 0)
    plsc.subcore_barrier()
    pltpu.sync_copy(acc.at[pl.ds(s * 624, 624)],
                    out_hbm.at[c, pl.ds(s * 624, 624)])

    @pl.when(s == 15)
    def _():
        pltpu.sync_copy(acc.at[pl.ds(9984, N_NODES - 9984)],
                        out_hbm.at[c, pl.ds(9984, N_NODES - 9984)])


@functools.partial(
    pl.kernel,
    out_type=jax.ShapeDtypeStruct((N_LOOKUP_PAD, TPAD), jnp.float32),
    mesh=_SC_MESH,
    compiler_params=pltpu.CompilerParams(use_tc_tiling_on_sc=False),
    scratch_types=[
        pltpu.VMEM((GCH, KCH), jnp.int32),
        pltpu.VMEM((KCH, TPAD), jnp.float32),
    ],
)
def _sc_embed_gather(table_hbm, idx_hbm, out_hbm, idx_v, rows_v):
    c = lax.axis_index("c")
    s = lax.axis_index("s")
    wid = c * 16 + s
    pltpu.sync_copy(idx_hbm.at[wid], idx_v)

    def body(j, carry):
        pltpu.sync_copy(table_hbm.at[idx_v.at[j]], rows_v)
        pltpu.sync_copy(rows_v, out_hbm.at[pl.ds(wid * LPW + j * KCH, KCH)])
        return carry

    lax.fori_loop(0, GCH, body, 0)


def _pad_table(tt):
    def kern(x_ref, o_ref):
        xt = jnp.transpose(x_ref[...], (1, 0))
        o_ref[...] = jnp.concatenate(
            [xt, jnp.zeros((xt.shape[0], TPAD - EMB), jnp.float32)], axis=1)

    return pl.pallas_call(
        kern,
        grid=(98,),
        in_specs=[pl.BlockSpec((EMB, 512), lambda i: (0, i))],
        out_specs=pl.BlockSpec((512, TPAD), lambda i: (i, 0)),
        out_shape=jax.ShapeDtypeStruct((tt.shape[1], TPAD), jnp.float32),
    )(tt)

def _mm_bias(x, wt, b):
    def kern(x_ref, wt_ref, b_ref, o_ref):
        o_ref[...] = jnp.dot(x_ref[...], wt_ref[...],
                             preferred_element_type=jnp.float32) + b_ref[...]

    return pl.pallas_call(
        kern,
        out_shape=jax.ShapeDtypeStruct((x.shape[0], wt.shape[1]), jnp.float32),
    )(x, wt, b)


def _mm_scale(x, wt, dinv):
    def kern(x_ref, wt_ref, d_ref, o_ref):
        o_ref[...] = jnp.dot(x_ref[...], wt_ref[...],
                             preferred_element_type=jnp.float32) * d_ref[...]

    return pl.pallas_call(
        kern,
        out_shape=jax.ShapeDtypeStruct((x.shape[0], wt.shape[1]), jnp.float32),
    )(x, wt, dinv)


def _dinv_col(deg_t):
    def kern(d_ref, o_ref):
        o_ref[...] = lax.rsqrt(1.0 + jnp.sum(d_ref[...], axis=1, keepdims=True))

    return pl.pallas_call(
        kern,
        out_shape=jax.ShapeDtypeStruct((deg_t.shape[0], 1), jnp.float32),
    )(deg_t)


def _epilogue(p, y, dinv, b, elu):
    def kern(p_ref, y_ref, d_ref, b_ref, o_ref):
        t = (p_ref[0] + p_ref[1] + y_ref[...]) * d_ref[...] + b_ref[...]
        if elu:
            t = jnp.where(t > 0, t, jnp.exp(jnp.minimum(t, 0.0)) - 1.0)
        o_ref[...] = t

    return pl.pallas_call(
        kern,
        out_shape=jax.ShapeDtypeStruct(y.shape, jnp.float32),
    )(p, y, dinv, b)


def _lstm_scan_kernel(xp, w, Bn, T):
    def kern(xp_ref, w_ref, hs_ref):
        w_v = w_ref[...]

        def step(t, carry):
            h, c = carry
            g = xp_ref[t] + jnp.dot(h, w_v, preferred_element_type=jnp.float32)
            i = jax.nn.sigmoid(g[:, 0:300])
            f = jax.nn.sigmoid(g[:, 300:600])
            gg = jnp.tanh(g[:, 600:900])
            o = jax.nn.sigmoid(g[:, 900:1200])
            c = f * c + i * gg
            h = o * jnp.tanh(c)
            hs_ref[t] = h
            return (h, c)

        z = jnp.zeros((Bn, 2 * HID), jnp.float32)
        lax.fori_loop(0, T, step, (z, z))

    return pl.pallas_call(
        kern,
        out_shape=jax.ShapeDtypeStruct((T, Bn, 2 * HID), jnp.float32),
    )(xp, w)


def _lstm_layer(p, layer, x, lens):
    Bn, T, _ = x.shape
    H = HID
    t = jnp.arange(T)
    idx = lens[:, None] - 1 - t[None, :]
    valid = (idx >= 0).astype(x.dtype)
    idxc = jnp.clip(idx, 0, T - 1)
    xb = jnp.take_along_axis(x, idxc[:, :, None], axis=1) * valid[:, :, None]

    wih_f = p['lstm_wih_%df' % layer].T
    wih_b = p['lstm_wih_%db' % layer].T
    whh_f = p['lstm_whh_%df' % layer].T
    whh_b = p['lstm_whh_%db' % layer].T
    bias_f = p['lstm_bih_%df' % layer] + p['lstm_bhh_%df' % layer]
    bias_b = p['lstm_bih_%db' % layer] + p['lstm_bhh_%db' % layer]

    xpf = (x @ wih_f + bias_f).reshape(Bn, T, 4, H)
    xpb = (xb @ wih_b + bias_b).reshape(Bn, T, 4, H)
    xp = jnp.concatenate([xpf, xpb], axis=-1).reshape(Bn, T, 8 * H)
    xp = jnp.swapaxes(xp, 0, 1)

    w = jnp.zeros((2 * H, 8 * H), jnp.float32)
    for k in range(4):
        w = w.at[:H, k * 2 * H:k * 2 * H + H].set(whh_f[:, k * H:(k + 1) * H])
        w = w.at[H:, k * 2 * H + H:(k + 1) * 2 * H].set(whh_b[:, k * H:(k + 1) * H])

    hs = _lstm_scan_kernel(xp, w, Bn, T)
    fwd = jnp.swapaxes(hs[:, :, :H], 0, 1)
    bwd = jnp.swapaxes(hs[:, :, H:], 0, 1)
    bwd = jnp.take_along_axis(bwd, idxc[:, :, None], axis=1) * valid[:, :, None]
    mask = (t[None, :] < lens[:, None]).astype(hs.dtype)[:, :, None]
    return jnp.concatenate([fwd * mask, bwd * mask], axis=-1)


def _bilstm(p, x, lens):
    h = x
    for layer in (0, 1):
        h = _lstm_layer(p, layer, h, lens)
    return h


def _encoder(p, pos_table, sent_pos_table, x, sent_x, embed, emb_flat):
    Bn, Sn, Ln = sent_x.shape
    BS = Bn * Sn
    tokens = sent_x.reshape(BS, Ln)
    sentlen = jnp.sum(tokens != 0, axis=-1)
    pos_idx = jnp.arange(1, Ln + 1)[None, :]
    posmask = (pos_idx <= sentlen[:, None]).astype(jnp.float32)
    pos_emb = posmask[:, :, None] * pos_table[1:Ln + 1][None, :, :]
    conv_in = emb_flat + pos_emb
    feats = []
    for h in range(2, 8):
        w = p['conv_w_%d' % h]
        bb = p['conv_b_%d' % h]
        P = Ln - h + 1
        acc = None
        for dh in range(h):
            term = jnp.einsum('bpe,oe->bpo', conv_in[:, dh:dh + P, :],
                              w[:, 0, dh, :])
            acc = term if acc is None else acc + term
        out = jax.nn.relu(acc + bb[None, None, :])
        feats.append(jnp.max(out, axis=1))
    ngram = jnp.concatenate(feats, axis=1)
    spos = jnp.tile(sent_pos_table[:Sn], (Bn, 1))
    cnn_feature = (ngram + spos) @ p['cnn_proj_w'].T + p['cnn_proj_b']
    cnn_feature = cnn_feature.reshape(Bn, Sn, HID)
    glen = jnp.sum(sent_x[:, :, 0] != 0, axis=-1)
    lstm_out = _bilstm(p, cnn_feature, glen)
    lstm_feature = lstm_out @ p['lstm_proj_w'].T + p['lstm_proj_b']
    sent_feat = jnp.concatenate([cnn_feature, lstm_feature], axis=-1)
    xcat = jnp.concatenate([embed, sent_feat], axis=1)
    return xcat.reshape(-1, EMB), embed


def kernel(x, sent_x, edge_index, params, pos_table, sent_pos_table):
    p = params

    src_r = edge_index[0].reshape(NWORKERS, NCH, KCH)
    dst_r = edge_index[1].reshape(NWORKERS, NCH, KCH)
    zero_deg = jnp.zeros((N_NODES,), jnp.float32)
    zero_rows = jnp.zeros((N_NODES, DPAD), jnp.float32)

    deg_part = _sc_degree(dst_r, zero_deg)
    dinv = _dinv_col(deg_part.T)

    tpad = _pad_table(p['embed'].T)
    all_idx = jnp.concatenate([
        x.reshape(-1), sent_x.reshape(-1),
        jnp.arange(N_LOOKUP_PAD - N_LOOKUP, dtype=jnp.int32)])
    g = _sc_embed_gather(tpad, all_idx.reshape(NWORKERS, GCH, KCH))
    embed = g[:9600, :EMB].reshape(x.shape[0], x.shape[1], EMB)
    emb_flat = g[9600:N_LOOKUP, :EMB].reshape(-1, sent_x.shape[2], EMB)

    xcat, embed = _encoder(p, pos_table, sent_pos_table, x, sent_x,
                           embed, emb_flat)

    h0 = _mm_bias(xcat, p['gc0_w'].T, p['gc0_b'][None, :])

    w1t = jnp.zeros((HID, DPAD), jnp.float32).at[:, :HID].set(p['gc1_w'].T)
    w2t = jnp.zeros((DPAD, DPAD), jnp.float32).at[:HID, :HID].set(p['gc2_w'].T)
    b1 = jnp.zeros((1, DPAD), jnp.float32).at[0, :HID].set(p['gc1_b'])
    b2 = jnp.zeros((1, DPAD), jnp.float32).at[0, :HID].set(p['gc2_b'])

    y1 = _mm_scale(h0, w1t, dinv)
    p1 = _sc_edge_prop(src_r, dst_r, y1, zero_rows)
    h1 = _epilogue(p1, y1, dinv, b1, elu=True)

    y2 = _mm_scale(h1, w2t, dinv)
    p2 = _sc_edge_prop(src_r, dst_r, y2, zero_rows)
    out = _epilogue(p2, y2, dinv, b2, elu=False)

    return out[:, :HID], embed


# speedup vs baseline: 1.0351x; 1.0030x over previous
import functools

import jax
import jax.numpy as jnp
from jax import lax
from jax.experimental import pallas as pl
from jax.experimental.pallas import tpu as pltpu
from jax.experimental.pallas import tpu_sc as plsc

EMB = 300
HID = 150
N_NODES = 10000
N_EDGES = 320000
DPAD = 152
NWORKERS = 32
EPW = N_EDGES // NWORKERS
KCH = 80
NCH = EPW // KCH
ECH = 96
ENCH = 105
EPWP = ECH * ENCH
N_ACC = N_NODES + 8

TPAD = 304
N_LOOKUP = 9600 + 20000
GCH = 12
N_LOOKUP_PAD = NWORKERS * GCH * KCH
LPW = GCH * KCH

_SC_MESH = plsc.VectorSubcoreMesh(core_axis_name="c", subcore_axis_name="s")


@functools.partial(
    pl.kernel,
    out_type=jax.ShapeDtypeStruct((2, N_NODES), jnp.float32),
    mesh=_SC_MESH,
    compiler_params=pltpu.CompilerParams(use_tc_tiling_on_sc=False),
    scratch_types=[
        pltpu.VMEM((NCH, KCH), jnp.int32),
        pltpu.VMEM((KCH,), jnp.float32),
        pltpu.VMEM_SHARED((N_NODES,), jnp.float32),
    ],
)
def _sc_degree(dst_hbm, zero_hbm, out_hbm, dst_v, ones_v, acc):
    c = lax.axis_index("c")
    s = lax.axis_index("s")
    wid = c * 16 + s

    @pl.when(s == 0)
    def _():
        pltpu.sync_copy(zero_hbm, acc)

    pltpu.sync_copy(dst_hbm.at[wid], dst_v)

    def fill(i, carry):
        ones_v[pl.ds(i * 16, 16)] = jnp.ones((16,), jnp.float32)
        return carry

    lax.fori_loop(0, KCH // 16, fill, 0)
    plsc.subcore_barrier()

    def body(j, carry):
        pltpu.sync_copy(ones_v, acc.at[dst_v.at[j]], add=True)
        return carry

    lax.fori_loop(0, NCH, body, 0)
    plsc.subcore_barrier()

    @pl.when(s == 0)
    def _():
        pltpu.sync_copy(acc, out_hbm.at[c])


@functools.partial(
    pl.kernel,
    out_type=jax.ShapeDtypeStruct((2, N_NODES, DPAD), jnp.float32),
    mesh=_SC_MESH,
    compiler_params=pltpu.CompilerParams(use_tc_tiling_on_sc=False),
    scratch_types=[
        pltpu.VMEM((ENCH, ECH), jnp.int32),
        pltpu.VMEM((ENCH, ECH), jnp.int32),
        pltpu.VMEM((ECH, DPAD), jnp.float32),
        pltpu.VMEM_SHARED((N_ACC, DPAD), jnp.float32),
    ],
)
def _sc_edge_prop(src_hbm, dst_hbm, y_hbm, zero_hbm, out_hbm,
                  src_v, dst_v, rows_v, acc):
    c = lax.axis_index("c")
    s = lax.axis_index("s")
    wid = c * 16 + s

    @pl.when(s == 0)
    def _():
        pltpu.sync_copy(zero_hbm, acc)

    pltpu.sync_copy(src_hbm.at[wid], src_v)
    pltpu.sync_copy(dst_hbm.at[wid], dst_v)
    plsc.subcore_barrier()

    def body(j, carry):
        pltpu.sync_copy(y_hbm.at[src_v.at[j]], rows_v)
        pltpu.sync_copy(rows_v, acc.at[dst_v.at[j]], add=True)
        return carry

    lax.fori_loop(0, ENCH, body, 0)
    plsc.subcore_barrier()
    pltpu.sync_copy(acc.at[pl.ds(s * 624, 624)],
                    out_hbm.at[c, pl.ds(s * 624, 624)])

    @pl.when(s == 15)
    def _():
        pltpu.sync_copy(acc.at[pl.ds(9984, N_NODES - 9984)],
                        out_hbm.at[c, pl.ds(9984, N_NODES - 9984)])


@functools.partial(
    pl.kernel,
    out_type=jax.ShapeDtypeStruct((N_LOOKUP_PAD, TPAD), jnp.float32),
    mesh=_SC_MESH,
    compiler_params=pltpu.CompilerParams(use_tc_tiling_on_sc=False),
    scratch_types=[
        pltpu.VMEM((GCH, KCH), jnp.int32),
        pltpu.VMEM((KCH, TPAD), jnp.float32),
    ],
)
def _sc_embed_gather(table_hbm, idx_hbm, out_hbm, idx_v, rows_v):
    c = lax.axis_index("c")
    s = lax.axis_index("s")
    wid = c * 16 + s
    pltpu.sync_copy(idx_hbm.at[wid], idx_v)

    def body(j, carry):
        pltpu.sync_copy(table_hbm.at[idx_v.at[j]], rows_v)
        pltpu.sync_copy(rows_v, out_hbm.at[pl.ds(wid * LPW + j * KCH, KCH)])
        return carry

    lax.fori_loop(0, GCH, body, 0)


def _pad_table(tt):
    def kern(x_ref, o_ref):
        xt = jnp.transpose(x_ref[...], (1, 0))
        o_ref[...] = jnp.concatenate(
            [xt, jnp.zeros((xt.shape[0], TPAD - EMB), jnp.float32)], axis=1)

    return pl.pallas_call(
        kern,
        grid=(98,),
        in_specs=[pl.BlockSpec((EMB, 512), lambda i: (0, i))],
        out_specs=pl.BlockSpec((512, TPAD), lambda i: (i, 0)),
        out_shape=jax.ShapeDtypeStruct((tt.shape[1], TPAD), jnp.float32),
    )(tt)

def _mm_bias(x, wt, b):
    def kern(x_ref, wt_ref, b_ref, o_ref):
        o_ref[...] = jnp.dot(x_ref[...], wt_ref[...],
                             preferred_element_type=jnp.float32) + b_ref[...]

    return pl.pallas_call(
        kern,
        out_shape=jax.ShapeDtypeStruct((x.shape[0], wt.shape[1]), jnp.float32),
    )(x, wt, b)


def _mm_scale(x, wt, dinv):
    def kern(x_ref, wt_ref, d_ref, o_ref):
        o_ref[...] = jnp.dot(x_ref[...], wt_ref[...],
                             preferred_element_type=jnp.float32) * d_ref[...]

    return pl.pallas_call(
        kern,
        out_shape=jax.ShapeDtypeStruct((x.shape[0], wt.shape[1]), jnp.float32),
    )(x, wt, dinv)


def _dinv_col(deg_t):
    def kern(d_ref, o_ref):
        o_ref[...] = lax.rsqrt(1.0 + jnp.sum(d_ref[...], axis=1, keepdims=True))

    return pl.pallas_call(
        kern,
        out_shape=jax.ShapeDtypeStruct((deg_t.shape[0], 1), jnp.float32),
    )(deg_t)


def _epilogue(p, y, dinv, b, elu):
    def kern(p_ref, y_ref, d_ref, b_ref, o_ref):
        t = (p_ref[0] + p_ref[1] + y_ref[...]) * d_ref[...] + b_ref[...]
        if elu:
            t = jnp.where(t > 0, t, jnp.exp(jnp.minimum(t, 0.0)) - 1.0)
        o_ref[...] = t

    return pl.pallas_call(
        kern,
        out_shape=jax.ShapeDtypeStruct(y.shape, jnp.float32),
    )(p, y, dinv, b)


def _lstm_scan_kernel(xp, w, Bn, T):
    def kern(xp_ref, w_ref, hs_ref):
        w_v = w_ref[...]

        def step(t, carry):
            h, c = carry
            g = xp_ref[t] + jnp.dot(h, w_v, preferred_element_type=jnp.float32)
            i = jax.nn.sigmoid(g[:, 0:300])
            f = jax.nn.sigmoid(g[:, 300:600])
            gg = jnp.tanh(g[:, 600:900])
            o = jax.nn.sigmoid(g[:, 900:1200])
            c = f * c + i * gg
            h = o * jnp.tanh(c)
            hs_ref[t] = h
            return (h, c)

        z = jnp.zeros((Bn, 2 * HID), jnp.float32)
        lax.fori_loop(0, T, step, (z, z))

    return pl.pallas_call(
        kern,
        out_shape=jax.ShapeDtypeStruct((T, Bn, 2 * HID), jnp.float32),
    )(xp, w)


def _lstm_layer(p, layer, x, lens):
    Bn, T, _ = x.shape
    H = HID
    t = jnp.arange(T)
    idx = lens[:, None] - 1 - t[None, :]
    valid = (idx >= 0).astype(x.dtype)
    idxc = jnp.clip(idx, 0, T - 1)
    xb = jnp.take_along_axis(x, idxc[:, :, None], axis=1) * valid[:, :, None]

    wih_f = p['lstm_wih_%df' % layer].T
    wih_b = p['lstm_wih_%db' % layer].T
    whh_f = p['lstm_whh_%df' % layer].T
    whh_b = p['lstm_whh_%db' % layer].T
    bias_f = p['lstm_bih_%df' % layer] + p['lstm_bhh_%df' % layer]
    bias_b = p['lstm_bih_%db' % layer] + p['lstm_bhh_%db' % layer]

    xpf = (x @ wih_f + bias_f).reshape(Bn, T, 4, H)
    xpb = (xb @ wih_b + bias_b).reshape(Bn, T, 4, H)
    xp = jnp.concatenate([xpf, xpb], axis=-1).reshape(Bn, T, 8 * H)
    xp = jnp.swapaxes(xp, 0, 1)

    w = jnp.zeros((2 * H, 8 * H), jnp.float32)
    for k in range(4):
        w = w.at[:H, k * 2 * H:k * 2 * H + H].set(whh_f[:, k * H:(k + 1) * H])
        w = w.at[H:, k * 2 * H + H:(k + 1) * 2 * H].set(whh_b[:, k * H:(k + 1) * H])

    hs = _lstm_scan_kernel(xp, w, Bn, T)
    fwd = jnp.swapaxes(hs[:, :, :H], 0, 1)
    bwd = jnp.swapaxes(hs[:, :, H:], 0, 1)
    bwd = jnp.take_along_axis(bwd, idxc[:, :, None], axis=1) * valid[:, :, None]
    mask = (t[None, :] < lens[:, None]).astype(hs.dtype)[:, :, None]
    return jnp.concatenate([fwd * mask, bwd * mask], axis=-1)


def _bilstm(p, x, lens):
    h = x
    for layer in (0, 1):
        h = _lstm_layer(p, layer, h, lens)
    return h


def _encoder(p, pos_table, sent_pos_table, x, sent_x, embed, emb_flat):
    Bn, Sn, Ln = sent_x.shape
    BS = Bn * Sn
    tokens = sent_x.reshape(BS, Ln)
    sentlen = jnp.sum(tokens != 0, axis=-1)
    pos_idx = jnp.arange(1, Ln + 1)[None, :]
    posmask = (pos_idx <= sentlen[:, None]).astype(jnp.float32)
    pos_emb = posmask[:, :, None] * pos_table[1:Ln + 1][None, :, :]
    conv_in = emb_flat + pos_emb
    feats = []
    for h in range(2, 8):
        w = p['conv_w_%d' % h]
        bb = p['conv_b_%d' % h]
        P = Ln - h + 1
        acc = None
        for dh in range(h):
            term = jnp.einsum('bpe,oe->bpo', conv_in[:, dh:dh + P, :],
                              w[:, 0, dh, :])
            acc = term if acc is None else acc + term
        out = jax.nn.relu(acc + bb[None, None, :])
        feats.append(jnp.max(out, axis=1))
    ngram = jnp.concatenate(feats, axis=1)
    spos = jnp.tile(sent_pos_table[:Sn], (Bn, 1))
    cnn_feature = (ngram + spos) @ p['cnn_proj_w'].T + p['cnn_proj_b']
    cnn_feature = cnn_feature.reshape(Bn, Sn, HID)
    glen = jnp.sum(sent_x[:, :, 0] != 0, axis=-1)
    lstm_out = _bilstm(p, cnn_feature, glen)
    lstm_feature = lstm_out @ p['lstm_proj_w'].T + p['lstm_proj_b']
    sent_feat = jnp.concatenate([cnn_feature, lstm_feature], axis=-1)
    xcat = jnp.concatenate([embed, sent_feat], axis=1)
    return xcat.reshape(-1, EMB), embed


def kernel(x, sent_x, edge_index, params, pos_table, sent_pos_table):
    p = params

    dst_r = edge_index[1].reshape(NWORKERS, NCH, KCH)
    zero_deg = jnp.zeros((N_NODES,), jnp.float32)
    zero_rows = jnp.zeros((N_ACC, DPAD), jnp.float32)

    npadw = EPWP - EPW
    dummy_src = (jnp.arange(NWORKERS * npadw, dtype=jnp.int32)
                 % N_NODES).reshape(NWORKERS, npadw)
    dummy_dst = (N_NODES + jnp.arange(NWORKERS * npadw, dtype=jnp.int32)
                 % (N_ACC - N_NODES)).reshape(NWORKERS, npadw)
    src_p = jnp.concatenate(
        [edge_index[0].reshape(NWORKERS, EPW), dummy_src],
        axis=1).reshape(NWORKERS, ENCH, ECH)
    dst_p = jnp.concatenate(
        [edge_index[1].reshape(NWORKERS, EPW), dummy_dst],
        axis=1).reshape(NWORKERS, ENCH, ECH)

    deg_part = _sc_degree(dst_r, zero_deg)
    dinv = _dinv_col(deg_part.T)

    tpad = _pad_table(p['embed'].T)
    all_idx = jnp.concatenate([
        x.reshape(-1), sent_x.reshape(-1),
        jnp.arange(N_LOOKUP_PAD - N_LOOKUP, dtype=jnp.int32)])
    g = _sc_embed_gather(tpad, all_idx.reshape(NWORKERS, GCH, KCH))
    embed = g[:9600, :EMB].reshape(x.shape[0], x.shape[1], EMB)
    emb_flat = g[9600:N_LOOKUP, :EMB].reshape(-1, sent_x.shape[2], EMB)

    xcat, embed = _encoder(p, pos_table, sent_pos_table, x, sent_x,
                           embed, emb_flat)

    h0 = _mm_bias(xcat, p['gc0_w'].T, p['gc0_b'][None, :])

    w1t = jnp.zeros((HID, DPAD), jnp.float32).at[:, :HID].set(p['gc1_w'].T)
    w2t = jnp.zeros((DPAD, DPAD), jnp.float32).at[:HID, :HID].set(p['gc2_w'].T)
    b1 = jnp.zeros((1, DPAD), jnp.float32).at[0, :HID].set(p['gc1_b'])
    b2 = jnp.zeros((1, DPAD), jnp.float32).at[0, :HID].set(p['gc2_b'])

    y1 = _mm_scale(h0, w1t, dinv)
    p1 = _sc_edge_prop(src_p, dst_p, y1, zero_rows)
    h1 = _epilogue(p1, y1, dinv, b1, elu=True)

    y2 = _mm_scale(h1, w2t, dinv)
    p2 = _sc_edge_prop(src_p, dst_p, y2, zero_rows)
    out = _epilogue(p2, y2, dinv, b2, elu=False)

    return out[:, :HID], embed
